# baseline (speedup 1.0000x reference)
import functools

import jax
import jax.numpy as jnp
from jax import lax
from jax.experimental import pallas as pl
from jax.experimental.pallas import tpu as pltpu
from jax.experimental.pallas import tpu_sc as plsc

D = 768
H = 2048
E = 8
K = 2
T = 2048

BM = 256
NB = (T * K) // BM
ITEMS = NB + E - 1
HC = 4
HCS = H // HC

NCORES = 2
NSUB = 16
NW = NCORES * NSUB
APT = (T * K) // NSUB
APW = (T * K) // NW
TPW = T // NW


def _k1_body(x_ref, gw_ref, sw1_ref, sw3_ref, sw2_ref,
             idx_ref, wgt_ref, ys_ref):
    hc = pl.program_id(1)
    x = x_ref[...]

    @pl.when(hc == 0)
    def _gate():
        logits = lax.dot_general(x, gw_ref[...],
                                 (((1,), (1,)), ((), ())),
                                 preferred_element_type=jnp.float32)
        col = lax.broadcasted_iota(jnp.int32, logits.shape, 1)
        logits = jnp.where(col < E, logits, -jnp.inf)
        m = jnp.max(logits, axis=-1, keepdims=True)
        p = jnp.exp(logits - m)
        s = p / jnp.sum(p, axis=-1, keepdims=True)
        i1 = jnp.argmax(s, axis=-1).astype(jnp.int32)
        m1 = jnp.max(s, axis=-1, keepdims=True)
        s2 = jnp.where(col == i1[:, None], -1.0, s)
        i2 = jnp.argmax(s2, axis=-1).astype(jnp.int32)
        m2 = jnp.max(s2, axis=-1, keepdims=True)
        denom = m1 + m2 + 1e-20
        idx_ref[...] = jnp.where(col == 0, i1[:, None],
                                 jnp.where(col == 1, i2[:, None], 0))
        wgt_ref[...] = jnp.where(col == 0, m1 / denom,
                                 jnp.where(col == 1, m2 / denom, 0.0))

    xb = x.astype(jnp.bfloat16)
    a = lax.dot_general(xb, sw1_ref[...], (((1,), (1,)), ((), ())),
                        preferred_element_type=jnp.float32)
    g = lax.dot_general(xb, sw3_ref[...], (((1,), (1,)), ((), ())),
                        preferred_element_type=jnp.float32)
    u = (a * jax.nn.sigmoid(a)) * g
    part = lax.dot_general(u.astype(jnp.bfloat16), sw2_ref[...],
                           (((1,), (1,)), ((), ())),
                           preferred_element_type=jnp.float32)

    @pl.when(hc == 0)
    def _init():
        ys_ref[...] = part

    @pl.when(hc != 0)
    def _acc():
        ys_ref[...] += part


def _gating_shared(h, gw_pad, sw1b, sw3b, sw2b):
    grid = (T // BM, HC)
    return pl.pallas_call(
        _k1_body,
        grid=grid,
        in_specs=[
            pl.BlockSpec((BM, D), lambda b, hc: (b, 0)),
            pl.BlockSpec((128, D), lambda b, hc: (0, 0)),
            pl.BlockSpec((HCS, D), lambda b, hc: (hc, 0)),
            pl.BlockSpec((HCS, D), lambda b, hc: (hc, 0)),
            pl.BlockSpec((D, HCS), lambda b, hc: (0, hc)),
        ],
        out_specs=[
            pl.BlockSpec((BM, 128), lambda b, hc: (b, 0)),
            pl.BlockSpec((BM, 128), lambda b, hc: (b, 0)),
            pl.BlockSpec((BM, D), lambda b, hc: (b, 0)),
        ],
        out_shape=[
            jax.ShapeDtypeStruct((T, 128), jnp.int32),
            jax.ShapeDtypeStruct((T, 128), jnp.float32),
            jax.ShapeDtypeStruct((T, D), jnp.float32),
        ],
    )(h, gw_pad, sw1b, sw3b, sw2b)


def _k2_body(e_hbm, x_hbm, xs_hbm, pos_hbm, cnt_hbm,
             ev, posq, posv, cntv, rows_v, sem):
    cid = lax.axis_index("c")
    sid = lax.axis_index("s")
    lane = lax.iota(jnp.int32, 16)

    pltpu.sync_copy(e_hbm, ev)
    my_first_vreg = sid * (APT // 16)

    def _hstep(kk, carry):
        counts, prefix = carry
        v = ev[pl.ds(kk * 16, 16)]
        add = jnp.zeros((16,), jnp.int32)
        for e in range(E):
            pc = jnp.sum((v == e).astype(jnp.int32))
            add = jnp.where(lane == e, add + pc, add)
        counts = counts + add
        prefix = prefix + jnp.where(jnp.full((16,), kk < my_first_vreg),
                                    add, 0)
        return counts, prefix

    counts, prefix = lax.fori_loop(
        0, (T * K) // 16, _hstep,
        (jnp.zeros((16,), jnp.int32), jnp.zeros((16,), jnp.int32)))
    total_excl = plsc.cumsum(counts) - counts
    run = total_excl + prefix

    for kk in range(APT // 16):
        v = ev[pl.ds(sid * APT + kk * 16, 16)]
        rank = jnp.zeros((16,), jnp.int32)
        base = jnp.zeros((16,), jnp.int32)
        tot = jnp.zeros((16,), jnp.int32)
        for e in range(E):
            m = v == e
            c = plsc.cumsum(m.astype(jnp.int32))
            rank = jnp.where(m, c - 1, rank)
            run_e = jnp.sum(jnp.where(lane == e, run, 0))
            base = jnp.where(m, run_e, base)
            pc = jnp.sum(m.astype(jnp.int32))
            tot = jnp.where(lane == e, pc, tot)
        p = base + rank
        posq[kk // 4, pl.ds((kk % 4) * 16, 16)] = p
        posv[pl.ds(kk * 16, 16)] = p
        run = run + tot

    @pl.when(cid == 0)
    def _store_pos():
        pltpu.sync_copy(posv, pos_hbm.at[pl.ds(sid * APT, APT)])

    @pl.when(jnp.logical_and(cid == 0, sid == 0))
    def _store_cnt():
        cntv[...] = counts
        pltpu.sync_copy(cntv, cnt_hbm)

    a0 = sid * APT + cid * APW
    t0 = a0 % T
    for c in range(APW // 64):
        pltpu.sync_copy(x_hbm.at[pl.ds(t0 + c * 64, 64)], rows_v)
        q = cid * (APW // 64) + c
        pltpu.async_copy(rows_v, xs_hbm.at[posq.at[q]], sem).wait()


def _route_sc(e_flat, h):
    mesh = plsc.VectorSubcoreMesh(core_axis_name="c", subcore_axis_name="s")
    kcall = pl.kernel(
        _k2_body,
        out_type=[
            jax.ShapeDtypeStruct((T * K, D), jnp.float32),
            jax.ShapeDtypeStruct((T * K,), jnp.int32),
            jax.ShapeDtypeStruct((16,), jnp.int32),
        ],
        mesh=mesh,
        scratch_types=[
            pltpu.VMEM((T * K,), jnp.int32),
            pltpu.VMEM((APT // 64, 64), jnp.int32),
            pltpu.VMEM((APT,), jnp.int32),
            pltpu.VMEM((16,), jnp.int32),
            pltpu.VMEM((64, D), jnp.float32),
            pltpu.SemaphoreType.DMA,
        ],
        compiler_params=pltpu.CompilerParams(needs_layout_passes=False),
    )
    return kcall(e_flat, h)


def _make_meta(counts):
    cnt = counts[:E]
    off = jnp.cumsum(cnt) - cnt
    tiles = jnp.where(cnt > 0, (off + cnt - 1) // BM - off // BM + 1, 0)
    cum = jnp.cumsum(tiles)
    cumx = cum - tiles
    i = jnp.arange(ITEMS, dtype=jnp.int32)
    eid = jnp.minimum(jnp.searchsorted(cum, i, side="right"), E - 1)
    eid = eid.astype(jnp.int32)
    j = i - cumx[eid]
    blk = off[eid] // BM + j
    start = jnp.maximum(off[eid], BM * blk)
    end = jnp.minimum(off[eid] + cnt[eid], BM * (blk + 1))
    valid = i < cum[E - 1]
    blk = jnp.where(valid, blk, NB - 1)
    start = jnp.where(valid, start, 0)
    end = jnp.where(valid, end, 0)
    return jnp.stack([blk.astype(jnp.int32), eid,
                      start.astype(jnp.int32), end.astype(jnp.int32)])


def _k3_body(m_ref, xs_ref, w1_ref, w3_ref, w2_ref, out_ref):
    i = pl.program_id(0)
    hc = pl.program_id(1)
    blk = m_ref[0, i]
    start = m_ref[2, i]
    end = m_ref[3, i]
    prev_blk = m_ref[0, jnp.maximum(i - 1, 0)]
    first = jnp.logical_and(hc == 0,
                            jnp.logical_or(i == 0, blk != prev_blk))

    @pl.when(first)
    def _init():
        out_ref[...] = jnp.zeros_like(out_ref)

    @pl.when(start < end)
    def _compute():
        xb = xs_ref[...].astype(jnp.bfloat16)
        a = lax.dot_general(xb, w1_ref[0], (((1,), (1,)), ((), ())),
                            preferred_element_type=jnp.float32)
        g = lax.dot_general(xb, w3_ref[0], (((1,), (1,)), ((), ())),
                            preferred_element_type=jnp.float32)
        u = (a * jax.nn.sigmoid(a)) * g
        r = lax.broadcasted_iota(jnp.int32, (BM, 1), 0) + BM * blk
        u = jnp.where(jnp.logical_and(r >= start, r < end), u, 0.0)
        out_ref[...] += lax.dot_general(u.astype(jnp.bfloat16), w2_ref[0],
                                        (((1,), (1,)), ((), ())),
                                        preferred_element_type=jnp.float32)


def _grouped_ffn(meta, xs, w1b, w3b, w2b):
    grid_spec = pltpu.PrefetchScalarGridSpec(
        num_scalar_prefetch=1,
        grid=(ITEMS, HC),
        in_specs=[
            pl.BlockSpec((BM, D), lambda i, hc, m: (m[0, i], 0)),
            pl.BlockSpec((1, HCS, D), lambda i, hc, m: (m[1, i], hc, 0)),
            pl.BlockSpec((1, HCS, D), lambda i, hc, m: (m[1, i], hc, 0)),
            pl.BlockSpec((1, D, HCS), lambda i, hc, m: (m[1, i], 0, hc)),
        ],
        out_specs=pl.BlockSpec((BM, D), lambda i, hc, m: (m[0, i], 0)),
    )
    return pl.pallas_call(
        _k3_body,
        grid_spec=grid_spec,
        out_shape=jax.ShapeDtypeStruct((T * K, D), jnp.float32),
    )(meta, xs, w1b, w3b, w2b)


def _k4_body(os_hbm, ys_hbm, pos_hbm, wgt_hbm, y_hbm,
             p0v, p1v, w0v, w1v, r0v, r1v, yv, sem):
    cid = lax.axis_index("c")
    sid = lax.axis_index("s")
    wid = sid * NCORES + cid
    t0 = wid * TPW
    lane = lax.iota(jnp.int32, 16)

    pltpu.sync_copy(pos_hbm.at[0, pl.ds(t0, TPW)], p0v)
    pltpu.sync_copy(pos_hbm.at[1, pl.ds(t0, TPW)], p1v)
    pltpu.sync_copy(wgt_hbm.at[0, pl.ds(t0, TPW)], w0v)
    pltpu.sync_copy(wgt_hbm.at[1, pl.ds(t0, TPW)], w1v)

    for c in range(TPW // 16):
        idx0 = p0v[pl.ds(c * 16, 16)]
        idx1 = p1v[pl.ds(c * 16, 16)]
        pltpu.async_copy(os_hbm.at[idx0], r0v, sem).wait()
        pltpu.async_copy(os_hbm.at[idx1], r1v, sem).wait()
        pltpu.sync_copy(ys_hbm.at[pl.ds(t0 + c * 16, 16)], yv)
        w0c = w0v[pl.ds(c * 16, 16)]
        w1c = w1v[pl.ds(c * 16, 16)]
        for j in range(16):
            w0s = jnp.sum(jnp.where(lane == j, w0c, 0.0))
            w1s = jnp.sum(jnp.where(lane == j, w1c, 0.0))

            def _dstep(dd, _, j=j, w0s=w0s, w1s=w1s):
                sl = pl.ds(dd * 16, 16)
                yv[j, sl] = (yv[j, sl] + w0s * r0v[j, sl]
                             + w1s * r1v[j, sl])
                return _

            lax.fori_loop(0, D // 16, _dstep, 0, unroll=8)
        pltpu.sync_copy(yv, y_hbm.at[pl.ds(t0 + c * 16, 16)])


def _combine_sc(out_sorted, ys, pos2, wgt2):
    mesh = plsc.VectorSubcoreMesh(core_axis_name="c", subcore_axis_name="s")
    kcall = pl.kernel(
        _k4_body,
        out_type=jax.ShapeDtypeStruct((T, D), jnp.float32),
        mesh=mesh,
        scratch_types=[
            pltpu.VMEM((TPW,), jnp.int32),
            pltpu.VMEM((TPW,), jnp.int32),
            pltpu.VMEM((TPW,), jnp.float32),
            pltpu.VMEM((TPW,), jnp.float32),
            pltpu.VMEM((16, D), jnp.float32),
            pltpu.VMEM((16, D), jnp.float32),
            pltpu.VMEM((16, D), jnp.float32),
            pltpu.SemaphoreType.DMA,
        ],
        compiler_params=pltpu.CompilerParams(needs_layout_passes=False),
    )
    return kcall(out_sorted, ys, pos2, wgt2)


def kernel(x, gate_weight, w1, w2, w3, sw1, sw2, sw3):
    b, s, d = x.shape
    h = x.reshape(-1, d)

    gw_pad = jnp.zeros((128, D), jnp.float32).at[:E].set(gate_weight)
    sw1b = sw1.astype(jnp.bfloat16)
    sw3b = sw3.astype(jnp.bfloat16)
    sw2b = sw2.astype(jnp.bfloat16)
    w1b = w1.astype(jnp.bfloat16)
    w3b = w3.astype(jnp.bfloat16)
    w2b = w2.astype(jnp.bfloat16)

    idx_out, wgt_out, ys = _gating_shared(h, gw_pad, sw1b, sw3b, sw2b)
    e_flat = idx_out[:, :K].T.reshape(T * K)
    wgt2 = wgt_out[:, :K].T

    xs, pos, counts = _route_sc(e_flat, h)
    meta = _make_meta(counts)
    out_sorted = _grouped_ffn(meta, xs, w1b, w3b, w2b)
    y = _combine_sc(out_sorted, ys, pos.reshape(K, T), wgt2)
    return y.reshape(b, s, d)


# speedup vs baseline: 1.2021x; 1.2021x over previous
import functools

import jax
import jax.numpy as jnp
from jax import lax
from jax.experimental import pallas as pl
from jax.experimental.pallas import tpu as pltpu
from jax.experimental.pallas import tpu_sc as plsc

D = 768
H = 2048
E = 8
K = 2
T = 2048

BM = 256
NB = (T * K) // BM
ITEMS = NB + E - 1
HC = 2
HCS = H // HC

NCORES = 2
NSUB = 16
NW = NCORES * NSUB
APT = (T * K) // NSUB
APW = (T * K) // NW
TPW = T // NW


def _k1_body(x_ref, gw_ref, sw1_ref, sw3_ref, sw2_ref,
             idx_ref, wgt_ref, ys_ref):
    hc = pl.program_id(1)
    x = x_ref[...]

    @pl.when(hc == 0)
    def _gate():
        logits = lax.dot_general(x, gw_ref[...],
                                 (((1,), (1,)), ((), ())),
                                 preferred_element_type=jnp.float32)
        col = lax.broadcasted_iota(jnp.int32, logits.shape, 1)
        logits = jnp.where(col < E, logits, -jnp.inf)
        m = jnp.max(logits, axis=-1, keepdims=True)
        p = jnp.exp(logits - m)
        s = p / jnp.sum(p, axis=-1, keepdims=True)
        i1 = jnp.argmax(s, axis=-1).astype(jnp.int32)
        m1 = jnp.max(s, axis=-1, keepdims=True)
        s2 = jnp.where(col == i1[:, None], -1.0, s)
        i2 = jnp.argmax(s2, axis=-1).astype(jnp.int32)
        m2 = jnp.max(s2, axis=-1, keepdims=True)
        denom = m1 + m2 + 1e-20
        idx_ref[...] = jnp.concatenate([i1[None, :], i2[None, :]], 0)
        wgt_ref[...] = jnp.concatenate([(m1 / denom).T, (m2 / denom).T], 0)

    xb = x.astype(jnp.bfloat16)
    a = lax.dot_general(xb, sw1_ref[...], (((1,), (1,)), ((), ())),
                        preferred_element_type=jnp.float32)
    g = lax.dot_general(xb, sw3_ref[...], (((1,), (1,)), ((), ())),
                        preferred_element_type=jnp.float32)
    u = (a * jax.nn.sigmoid(a)) * g
    part = lax.dot_general(u.astype(jnp.bfloat16), sw2_ref[...],
                           (((1,), (1,)), ((), ())),
                           preferred_element_type=jnp.float32)

    @pl.when(hc == 0)
    def _init():
        ys_ref[...] = part

    @pl.when(hc != 0)
    def _acc():
        ys_ref[...] += part


def _gating_shared(h, gw_pad, sw1b, sw3b, sw2b):
    grid = (T // BM, HC)
    return pl.pallas_call(
        _k1_body,
        grid=grid,
        in_specs=[
            pl.BlockSpec((BM, D), lambda b, hc: (b, 0)),
            pl.BlockSpec((128, D), lambda b, hc: (0, 0)),
            pl.BlockSpec((HCS, D), lambda b, hc: (hc, 0)),
            pl.BlockSpec((HCS, D), lambda b, hc: (hc, 0)),
            pl.BlockSpec((D, HCS), lambda b, hc: (0, hc)),
        ],
        out_specs=[
            pl.BlockSpec((K, BM), lambda b, hc: (0, b)),
            pl.BlockSpec((K, BM), lambda b, hc: (0, b)),
            pl.BlockSpec((BM, D), lambda b, hc: (b, 0)),
        ],
        out_shape=[
            jax.ShapeDtypeStruct((K, T), jnp.int32),
            jax.ShapeDtypeStruct((K, T), jnp.float32),
            jax.ShapeDtypeStruct((T, D), jnp.float32),
        ],
    )(h, gw_pad, sw1b, sw3b, sw2b)


def _k2_body(e_hbm, x_hbm, xs_hbm, pos_hbm, cnt_hbm,
             ev, posq, posv, cntv, rows_v, sem):
    cid = lax.axis_index("c")
    sid = lax.axis_index("s")
    lane = lax.iota(jnp.int32, 16)

    pltpu.sync_copy(e_hbm, ev)
    my_first_vreg = sid * (APT // 16)

    def _hstep(kk, carry):
        counts, prefix = carry
        v = ev[pl.ds(kk * 16, 16)]
        add = jnp.zeros((16,), jnp.int32)
        for e in range(E):
            pc = jnp.sum((v == e).astype(jnp.int32))
            add = jnp.where(lane == e, add + pc, add)
        counts = counts + add
        prefix = prefix + jnp.where(jnp.full((16,), kk < my_first_vreg),
                                    add, 0)
        return counts, prefix

    counts, prefix = lax.fori_loop(
        0, (T * K) // 16, _hstep,
        (jnp.zeros((16,), jnp.int32), jnp.zeros((16,), jnp.int32)))
    total_excl = plsc.cumsum(counts) - counts
    run = total_excl + prefix

    for kk in range(APT // 16):
        v = ev[pl.ds(sid * APT + kk * 16, 16)]
        rank = jnp.zeros((16,), jnp.int32)
        base = jnp.zeros((16,), jnp.int32)
        tot = jnp.zeros((16,), jnp.int32)
        for e in range(E):
            m = v == e
            c = plsc.cumsum(m.astype(jnp.int32))
            rank = jnp.where(m, c - 1, rank)
            run_e = jnp.sum(jnp.where(lane == e, run, 0))
            base = jnp.where(m, run_e, base)
            pc = jnp.sum(m.astype(jnp.int32))
            tot = jnp.where(lane == e, pc, tot)
        p = base + rank
        posq[kk // 4, pl.ds((kk % 4) * 16, 16)] = p
        posv[pl.ds(kk * 16, 16)] = p
        run = run + tot

    @pl.when(cid == 0)
    def _store_pos():
        pltpu.sync_copy(posv, pos_hbm.at[pl.ds(sid * APT, APT)])

    @pl.when(jnp.logical_and(cid == 0, sid == 0))
    def _store_cnt():
        cntv[...] = counts
        pltpu.sync_copy(cntv, cnt_hbm)

    a0 = sid * APT + cid * APW
    t0 = a0 % T
    for c in range(APW // 64):
        pltpu.sync_copy(x_hbm.at[pl.ds(t0 + c * 64, 64)], rows_v)
        q = cid * (APW // 64) + c
        pltpu.async_copy(rows_v, xs_hbm.at[posq.at[q]], sem).wait()


def _route_sc(e_flat, h):
    mesh = plsc.VectorSubcoreMesh(core_axis_name="c", subcore_axis_name="s")
    kcall = pl.kernel(
        _k2_body,
        out_type=[
            jax.ShapeDtypeStruct((T * K, D), jnp.float32),
            jax.ShapeDtypeStruct((T * K,), jnp.int32),
            jax.ShapeDtypeStruct((16,), jnp.int32),
        ],
        mesh=mesh,
        scratch_types=[
            pltpu.VMEM((T * K,), jnp.int32),
            pltpu.VMEM((APT // 64, 64), jnp.int32),
            pltpu.VMEM((APT,), jnp.int32),
            pltpu.VMEM((16,), jnp.int32),
            pltpu.VMEM((64, D), jnp.float32),
            pltpu.SemaphoreType.DMA,
        ],
        compiler_params=pltpu.CompilerParams(needs_layout_passes=False),
    )
    return kcall(e_flat, h)


def _make_meta(counts):
    cnt = counts[:E]
    off = jnp.cumsum(cnt) - cnt
    tiles = jnp.where(cnt > 0, (off + cnt - 1) // BM - off // BM + 1, 0)
    cum = jnp.cumsum(tiles)
    cumx = cum - tiles
    i = jnp.arange(ITEMS, dtype=jnp.int32)
    eid = jnp.minimum(jnp.searchsorted(cum, i, side="right"), E - 1)
    eid = eid.astype(jnp.int32)
    j = i - cumx[eid]
    blk = off[eid] // BM + j
    start = jnp.maximum(off[eid], BM * blk)
    end = jnp.minimum(off[eid] + cnt[eid], BM * (blk + 1))
    valid = i < cum[E - 1]
    blk = jnp.where(valid, blk, NB - 1)
    start = jnp.where(valid, start, 0)
    end = jnp.where(valid, end, 0)
    return jnp.stack([blk.astype(jnp.int32), eid,
                      start.astype(jnp.int32), end.astype(jnp.int32)])


def _k3_body(m_ref, xs_ref, w1_ref, w3_ref, w2_ref, out_ref):
    i = pl.program_id(0)
    hc = pl.program_id(1)
    blk = m_ref[0, i]
    start = m_ref[2, i]
    end = m_ref[3, i]
    prev_blk = m_ref[0, jnp.maximum(i - 1, 0)]
    first = jnp.logical_and(hc == 0,
                            jnp.logical_or(i == 0, blk != prev_blk))

    @pl.when(first)
    def _init():
        out_ref[...] = jnp.zeros_like(out_ref)

    @pl.when(start < end)
    def _compute():
        xb = xs_ref[...].astype(jnp.bfloat16)
        a = lax.dot_general(xb, w1_ref[0], (((1,), (1,)), ((), ())),
                            preferred_element_type=jnp.float32)
        g = lax.dot_general(xb, w3_ref[0], (((1,), (1,)), ((), ())),
                            preferred_element_type=jnp.float32)
        u = (a * jax.nn.sigmoid(a)) * g
        r = lax.broadcasted_iota(jnp.int32, (BM, 1), 0) + BM * blk
        u = jnp.where(jnp.logical_and(r >= start, r < end), u, 0.0)
        out_ref[...] += lax.dot_general(u.astype(jnp.bfloat16), w2_ref[0],
                                        (((1,), (1,)), ((), ())),
                                        preferred_element_type=jnp.float32)


def _grouped_ffn(meta, xs, w1b, w3b, w2b):
    grid_spec = pltpu.PrefetchScalarGridSpec(
        num_scalar_prefetch=1,
        grid=(ITEMS, HC),
        in_specs=[
            pl.BlockSpec((BM, D), lambda i, hc, m: (m[0, i], 0)),
            pl.BlockSpec((1, HCS, D), lambda i, hc, m: (m[1, i], hc, 0)),
            pl.BlockSpec((1, HCS, D), lambda i, hc, m: (m[1, i], hc, 0)),
            pl.BlockSpec((1, D, HCS), lambda i, hc, m: (m[1, i], 0, hc)),
        ],
        out_specs=pl.BlockSpec((BM, D), lambda i, hc, m: (m[0, i], 0)),
    )
    return pl.pallas_call(
        _k3_body,
        grid_spec=grid_spec,
        out_shape=jax.ShapeDtypeStruct((T * K, D), jnp.float32),
    )(meta, xs, w1b, w3b, w2b)


def _k4_body(os_hbm, ys_hbm, pos_hbm, wgt_hbm, y_hbm,
             p0v, p1v, w0v, w1v, r0v, r1v, ysv, yv, semA, semB):
    cid = lax.axis_index("c")
    sid = lax.axis_index("s")
    wid = sid * NCORES + cid
    t0 = wid * TPW
    lane = lax.iota(jnp.int32, 16)
    sems = (semA, semB)
    NCH = TPW // 16

    pltpu.sync_copy(pos_hbm.at[0, pl.ds(t0, TPW)], p0v)
    pltpu.sync_copy(pos_hbm.at[1, pl.ds(t0, TPW)], p1v)
    pltpu.sync_copy(wgt_hbm.at[0, pl.ds(t0, TPW)], w0v)
    pltpu.sync_copy(wgt_hbm.at[1, pl.ds(t0, TPW)], w1v)

    def _fire(c):
        buf = c % 2
        idx0 = p0v[pl.ds(c * 16, 16)]
        idx1 = p1v[pl.ds(c * 16, 16)]
        return (
            pltpu.async_copy(os_hbm.at[idx0], r0v.at[buf], sems[buf]),
            pltpu.async_copy(os_hbm.at[idx1], r1v.at[buf], sems[buf]),
            pltpu.async_copy(ys_hbm.at[pl.ds(t0 + c * 16, 16)],
                             ysv.at[buf], sems[buf]),
        )

    inflight = _fire(0)
    for c in range(NCH):
        buf = c % 2
        nxt = _fire(c + 1) if c + 1 < NCH else None
        for cp in inflight:
            cp.wait()
        inflight = nxt
        w0c = w0v[pl.ds(c * 16, 16)]
        w1c = w1v[pl.ds(c * 16, 16)]
        for j in range(16):
            w0s = jnp.sum(jnp.where(lane == j, w0c, 0.0))
            w1s = jnp.sum(jnp.where(lane == j, w1c, 0.0))

            def _dstep(dd, _, j=j, buf=buf, w0s=w0s, w1s=w1s):
                sl = pl.ds(dd * 16, 16)
                yv[j, sl] = (ysv[buf, j, sl] + w0s * r0v[buf, j, sl]
                             + w1s * r1v[buf, j, sl])
                return _

            lax.fori_loop(0, D // 16, _dstep, 0, unroll=8)
        pltpu.sync_copy(yv, y_hbm.at[pl.ds(t0 + c * 16, 16)])


def _combine_sc(out_sorted, ys, pos2, wgt2):
    mesh = plsc.VectorSubcoreMesh(core_axis_name="c", subcore_axis_name="s")
    kcall = pl.kernel(
        _k4_body,
        out_type=jax.ShapeDtypeStruct((T, D), jnp.float32),
        mesh=mesh,
        scratch_types=[
            pltpu.VMEM((TPW,), jnp.int32),
            pltpu.VMEM((TPW,), jnp.int32),
            pltpu.VMEM((TPW,), jnp.float32),
            pltpu.VMEM((TPW,), jnp.float32),
            pltpu.VMEM((2, 16, D), jnp.float32),
            pltpu.VMEM((2, 16, D), jnp.float32),
            pltpu.VMEM((2, 16, D), jnp.float32),
            pltpu.VMEM((16, D), jnp.float32),
            pltpu.SemaphoreType.DMA,
            pltpu.SemaphoreType.DMA,
        ],
        compiler_params=pltpu.CompilerParams(needs_layout_passes=False),
    )
    return kcall(out_sorted, ys, pos2, wgt2)


def kernel(x, gate_weight, w1, w2, w3, sw1, sw2, sw3):
    b, s, d = x.shape
    h = x.reshape(-1, d)

    gw_pad = jnp.zeros((128, D), jnp.float32).at[:E].set(gate_weight)
    sw1b = sw1.astype(jnp.bfloat16)
    sw3b = sw3.astype(jnp.bfloat16)
    sw2b = sw2.astype(jnp.bfloat16)
    w1b = w1.astype(jnp.bfloat16)
    w3b = w3.astype(jnp.bfloat16)
    w2b = w2.astype(jnp.bfloat16)

    eT, wgt2, ys = _gating_shared(h, gw_pad, sw1b, sw3b, sw2b)
    e_flat = eT.reshape(T * K)

    xs, pos, counts = _route_sc(e_flat, h)
    meta = _make_meta(counts)
    out_sorted = _grouped_ffn(meta, xs, w1b, w3b, w2b)
    y = _combine_sc(out_sorted, ys, pos.reshape(K, T), wgt2)
    return y.reshape(b, s, d)


# speedup vs baseline: 1.3141x; 1.0932x over previous
import functools

import jax
import jax.numpy as jnp
from jax import lax
from jax.experimental import pallas as pl
from jax.experimental.pallas import tpu as pltpu
from jax.experimental.pallas import tpu_sc as plsc

D = 768
H = 2048
E = 8
K = 2
T = 2048

BM = 256
NB = (T * K) // BM
ITEMS = NB + E - 1
HC = 2
HCS = H // HC

NCORES = 2
NSUB = 16
NW = NCORES * NSUB
APT = (T * K) // NSUB
APW = (T * K) // NW
TPW = T // NW


def _k1a_body(x_ref, gw_ref, idx_ref, wgt_ref):
    x = x_ref[...]
    logits = lax.dot_general(x, gw_ref[...],
                             (((1,), (1,)), ((), ())),
                             preferred_element_type=jnp.float32)
    col = lax.broadcasted_iota(jnp.int32, logits.shape, 1)
    logits = jnp.where(col < E, logits, -jnp.inf)
    m = jnp.max(logits, axis=-1, keepdims=True)
    p = jnp.exp(logits - m)
    s = p / jnp.sum(p, axis=-1, keepdims=True)
    i1 = jnp.argmax(s, axis=-1).astype(jnp.int32)
    m1 = jnp.max(s, axis=-1, keepdims=True)
    s2 = jnp.where(col == i1[:, None], -1.0, s)
    i2 = jnp.argmax(s2, axis=-1).astype(jnp.int32)
    m2 = jnp.max(s2, axis=-1, keepdims=True)
    denom = m1 + m2 + 1e-20
    idx_ref[...] = jnp.concatenate([i1[None, :], i2[None, :]], 0)
    wgt_ref[...] = jnp.concatenate([(m1 / denom).T, (m2 / denom).T], 0)


def _gating(h, gw_pad):
    return pl.pallas_call(
        _k1a_body,
        grid=(T // BM,),
        in_specs=[
            pl.BlockSpec((BM, D), lambda b: (b, 0)),
            pl.BlockSpec((128, D), lambda b: (0, 0)),
        ],
        out_specs=[
            pl.BlockSpec((K, BM), lambda b: (0, b)),
            pl.BlockSpec((K, BM), lambda b: (0, b)),
        ],
        out_shape=[
            jax.ShapeDtypeStruct((K, T), jnp.int32),
            jax.ShapeDtypeStruct((K, T), jnp.float32),
        ],
    )(h, gw_pad)


def _k1b_body(x_ref, sw1_ref, sw3_ref, sw2_ref, ys_ref):
    hc = pl.program_id(1)
    x = x_ref[...]
    a = lax.dot_general(x, sw1_ref[...], (((1,), (1,)), ((), ())),
                        preferred_element_type=jnp.float32)
    g = lax.dot_general(x, sw3_ref[...], (((1,), (1,)), ((), ())),
                        preferred_element_type=jnp.float32)
    u = (a * jax.nn.sigmoid(a)) * g
    part = lax.dot_general(u, sw2_ref[...], (((1,), (1,)), ((), ())),
                           preferred_element_type=jnp.float32)

    @pl.when(hc == 0)
    def _init():
        ys_ref[...] = part

    @pl.when(hc != 0)
    def _acc():
        ys_ref[...] += part


def _shared_ffn(h, sw1, sw3, sw2):
    return pl.pallas_call(
        _k1b_body,
        grid=(T // BM, HC),
        in_specs=[
            pl.BlockSpec((BM, D), lambda b, hc: (b, 0)),
            pl.BlockSpec((HCS, D), lambda b, hc: (hc, 0)),
            pl.BlockSpec((HCS, D), lambda b, hc: (hc, 0)),
            pl.BlockSpec((D, HCS), lambda b, hc: (0, hc)),
        ],
        out_specs=pl.BlockSpec((BM, D), lambda b, hc: (b, 0)),
        out_shape=jax.ShapeDtypeStruct((T, D), jnp.float32),
    )(h, sw1, sw3, sw2)


def _k2_body(e_hbm, x_hbm, xs_hbm, pos_hbm, cnt_hbm,
             ev, posq, posv, cntv, rows_v, sem):
    cid = lax.axis_index("c")
    sid = lax.axis_index("s")
    lane = lax.iota(jnp.int32, 16)

    pltpu.sync_copy(e_hbm, ev)
    my_first_vreg = sid * (APT // 16)

    def _hstep(kk, carry):
        counts, prefix = carry
        v = ev[pl.ds(kk * 16, 16)]
        add = jnp.zeros((16,), jnp.int32)
        for e in range(E):
            pc = jnp.sum((v == e).astype(jnp.int32))
            add = jnp.where(lane == e, add + pc, add)
        counts = counts + add
        prefix = prefix + jnp.where(jnp.full((16,), kk < my_first_vreg),
                                    add, 0)
        return counts, prefix

    counts, prefix = lax.fori_loop(
        0, (T * K) // 16, _hstep,
        (jnp.zeros((16,), jnp.int32), jnp.zeros((16,), jnp.int32)))
    total_excl = plsc.cumsum(counts) - counts
    run = total_excl + prefix

    for kk in range(APT // 16):
        v = ev[pl.ds(sid * APT + kk * 16, 16)]
        rank = jnp.zeros((16,), jnp.int32)
        base = jnp.zeros((16,), jnp.int32)
        tot = jnp.zeros((16,), jnp.int32)
        for e in range(E):
            m = v == e
            c = plsc.cumsum(m.astype(jnp.int32))
            rank = jnp.where(m, c - 1, rank)
            run_e = jnp.sum(jnp.where(lane == e, run, 0))
            base = jnp.where(m, run_e, base)
            pc = jnp.sum(m.astype(jnp.int32))
            tot = jnp.where(lane == e, pc, tot)
        p = base + rank
        posq[kk // 4, pl.ds((kk % 4) * 16, 16)] = p
        posv[pl.ds(kk * 16, 16)] = p
        run = run + tot

    @pl.when(cid == 0)
    def _store_pos():
        pltpu.sync_copy(posv, pos_hbm.at[pl.ds(sid * APT, APT)])

    @pl.when(jnp.logical_and(cid == 0, sid == 0))
    def _store_cnt():
        cntv[...] = counts
        pltpu.sync_copy(cntv, cnt_hbm)

    a0 = sid * APT + cid * APW
    t0 = a0 % T
    for c in range(APW // 64):
        pltpu.sync_copy(x_hbm.at[pl.ds(t0 + c * 64, 64)], rows_v)
        q = cid * (APW // 64) + c
        pltpu.async_copy(rows_v, xs_hbm.at[posq.at[q]], sem).wait()


def _route_sc(e_flat, h):
    mesh = plsc.VectorSubcoreMesh(core_axis_name="c", subcore_axis_name="s")
    kcall = pl.kernel(
        _k2_body,
        out_type=[
            jax.ShapeDtypeStruct((T * K, D), jnp.float32),
            jax.ShapeDtypeStruct((T * K,), jnp.int32),
            jax.ShapeDtypeStruct((16,), jnp.int32),
        ],
        mesh=mesh,
        scratch_types=[
            pltpu.VMEM((T * K,), jnp.int32),
            pltpu.VMEM((APT // 64, 64), jnp.int32),
            pltpu.VMEM((APT,), jnp.int32),
            pltpu.VMEM((16,), jnp.int32),
            pltpu.VMEM((64, D), jnp.float32),
            pltpu.SemaphoreType.DMA,
        ],
        compiler_params=pltpu.CompilerParams(needs_layout_passes=False),
    )
    return kcall(e_flat, h)


def _make_meta(counts):
    cnt = counts[:E]
    off = jnp.cumsum(cnt) - cnt
    tiles = jnp.where(cnt > 0, (off + cnt - 1) // BM - off // BM + 1, 0)
    cum = jnp.cumsum(tiles)
    cumx = cum - tiles
    i = jnp.arange(ITEMS, dtype=jnp.int32)
    eid = jnp.minimum(jnp.searchsorted(cum, i, side="right"), E - 1)
    eid = eid.astype(jnp.int32)
    j = i - cumx[eid]
    blk = off[eid] // BM + j
    start = jnp.maximum(off[eid], BM * blk)
    end = jnp.minimum(off[eid] + cnt[eid], BM * (blk + 1))
    valid = i < cum[E - 1]
    blk = jnp.where(valid, blk, NB - 1)
    start = jnp.where(valid, start, 0)
    end = jnp.where(valid, end, 0)
    return jnp.stack([blk.astype(jnp.int32), eid,
                      start.astype(jnp.int32), end.astype(jnp.int32)])


def _k3_body(m_ref, xs_ref, w1_ref, w3_ref, w2_ref, out_ref):
    i = pl.program_id(0)
    hc = pl.program_id(1)
    blk = m_ref[0, i]
    start = m_ref[2, i]
    end = m_ref[3, i]
    prev_blk = m_ref[0, jnp.maximum(i - 1, 0)]
    first = jnp.logical_and(hc == 0,
                            jnp.logical_or(i == 0, blk != prev_blk))

    @pl.when(first)
    def _init():
        out_ref[...] = jnp.zeros_like(out_ref)

    @pl.when(start < end)
    def _compute():
        xb = xs_ref[...]
        a = lax.dot_general(xb, w1_ref[0], (((1,), (1,)), ((), ())),
                            preferred_element_type=jnp.float32)
        g = lax.dot_general(xb, w3_ref[0], (((1,), (1,)), ((), ())),
                            preferred_element_type=jnp.float32)
        u = (a * jax.nn.sigmoid(a)) * g
        r = lax.broadcasted_iota(jnp.int32, (BM, 1), 0) + BM * blk
        u = jnp.where(jnp.logical_and(r >= start, r < end), u, 0.0)
        out_ref[...] += lax.dot_general(u, w2_ref[0],
                                        (((1,), (1,)), ((), ())),
                                        preferred_element_type=jnp.float32)


def _grouped_ffn(meta, xs, w1b, w3b, w2b):
    grid_spec = pltpu.PrefetchScalarGridSpec(
        num_scalar_prefetch=1,
        grid=(ITEMS, HC),
        in_specs=[
            pl.BlockSpec((BM, D), lambda i, hc, m: (m[0, i], 0)),
            pl.BlockSpec((1, HCS, D), lambda i, hc, m: (m[1, i], hc, 0)),
            pl.BlockSpec((1, HCS, D), lambda i, hc, m: (m[1, i], hc, 0)),
            pl.BlockSpec((1, D, HCS), lambda i, hc, m: (m[1, i], 0, hc)),
        ],
        out_specs=pl.BlockSpec((BM, D), lambda i, hc, m: (m[0, i], 0)),
    )
    return pl.pallas_call(
        _k3_body,
        grid_spec=grid_spec,
        out_shape=jax.ShapeDtypeStruct((T * K, D), jnp.float32),
    )(meta, xs, w1b, w3b, w2b)


def _k4_body(os_hbm, ys_hbm, pos_hbm, wgt_hbm, y_hbm,
             p0v, p1v, w0v, w1v, r0v, r1v, ysv, yv, semA, semB):
    cid = lax.axis_index("c")
    sid = lax.axis_index("s")
    wid = sid * NCORES + cid
    t0 = wid * TPW
    lane = lax.iota(jnp.int32, 16)
    sems = (semA, semB)
    NCH = TPW // 16

    pltpu.sync_copy(pos_hbm.at[0, pl.ds(t0, TPW)], p0v)
    pltpu.sync_copy(pos_hbm.at[1, pl.ds(t0, TPW)], p1v)
    pltpu.sync_copy(wgt_hbm.at[0, pl.ds(t0, TPW)], w0v)
    pltpu.sync_copy(wgt_hbm.at[1, pl.ds(t0, TPW)], w1v)

    def _fire(c):
        buf = c % 2
        idx0 = p0v[pl.ds(c * 16, 16)]
        idx1 = p1v[pl.ds(c * 16, 16)]
        return (
            pltpu.async_copy(os_hbm.at[idx0], r0v.at[buf], sems[buf]),
            pltpu.async_copy(os_hbm.at[idx1], r1v.at[buf], sems[buf]),
            pltpu.async_copy(ys_hbm.at[pl.ds(t0 + c * 16, 16)],
                             ysv.at[buf], sems[buf]),
        )

    inflight = _fire(0)
    for c in range(NCH):
        buf = c % 2
        nxt = _fire(c + 1) if c + 1 < NCH else None
        for cp in inflight:
            cp.wait()
        inflight = nxt
        w0c = w0v[pl.ds(c * 16, 16)]
        w1c = w1v[pl.ds(c * 16, 16)]
        for j in range(16):
            w0s = jnp.sum(jnp.where(lane == j, w0c, 0.0))
            w1s = jnp.sum(jnp.where(lane == j, w1c, 0.0))

            def _dstep(dd, _, j=j, buf=buf, w0s=w0s, w1s=w1s):
                sl = pl.ds(dd * 16, 16)
                yv[j, sl] = (ysv[buf, j, sl] + w0s * r0v[buf, j, sl]
                             + w1s * r1v[buf, j, sl])
                return _

            lax.fori_loop(0, D // 16, _dstep, 0, unroll=8)
        pltpu.sync_copy(yv, y_hbm.at[pl.ds(t0 + c * 16, 16)])


def _combine_sc(out_sorted, ys, pos2, wgt2):
    mesh = plsc.VectorSubcoreMesh(core_axis_name="c", subcore_axis_name="s")
    kcall = pl.kernel(
        _k4_body,
        out_type=jax.ShapeDtypeStruct((T, D), jnp.float32),
        mesh=mesh,
        scratch_types=[
            pltpu.VMEM((TPW,), jnp.int32),
            pltpu.VMEM((TPW,), jnp.int32),
            pltpu.VMEM((TPW,), jnp.float32),
            pltpu.VMEM((TPW,), jnp.float32),
            pltpu.VMEM((2, 16, D), jnp.float32),
            pltpu.VMEM((2, 16, D), jnp.float32),
            pltpu.VMEM((2, 16, D), jnp.float32),
            pltpu.VMEM((16, D), jnp.float32),
            pltpu.SemaphoreType.DMA,
            pltpu.SemaphoreType.DMA,
        ],
        compiler_params=pltpu.CompilerParams(needs_layout_passes=False),
    )
    return kcall(out_sorted, ys, pos2, wgt2)


def kernel(x, gate_weight, w1, w2, w3, sw1, sw2, sw3):
    b, s, d = x.shape
    h = x.reshape(-1, d)

    gw_pad = jnp.zeros((128, D), jnp.float32).at[:E].set(gate_weight)

    eT, wgt2 = _gating(h, gw_pad)
    e_flat = eT.reshape(T * K)

    xs, pos, counts = _route_sc(e_flat, h)
    ys = _shared_ffn(h, sw1, sw3, sw2)
    meta = _make_meta(counts)
    out_sorted = _grouped_ffn(meta, xs, w1, w3, w2)
    y = _combine_sc(out_sorted, ys, pos.reshape(K, T), wgt2)
    return y.reshape(b, s, d)


# speedup vs baseline: 1.3420x; 1.0212x over previous
import functools

import jax
import jax.numpy as jnp
from jax import lax
from jax.experimental import pallas as pl
from jax.experimental.pallas import tpu as pltpu
from jax.experimental.pallas import tpu_sc as plsc

D = 768
H = 2048
E = 8
K = 2
T = 2048

BM = 256
NB = (T * K) // BM
ITEMS = NB + E - 1
HC = 2
HCS = H // HC

NCORES = 2
NSUB = 16
NW = NCORES * NSUB
APT = (T * K) // NSUB
APW = (T * K) // NW
TPW = T // NW


def _k1a_body(x_ref, gw_ref, idx_ref, wgt_ref):
    x = x_ref[...]
    logits = lax.dot_general(x, gw_ref[...],
                             (((1,), (1,)), ((), ())),
                             preferred_element_type=jnp.float32)
    col = lax.broadcasted_iota(jnp.int32, logits.shape, 1)
    m = jnp.max(logits, axis=-1, keepdims=True)
    p = jnp.exp(logits - m)
    s = p / jnp.sum(p, axis=-1, keepdims=True)
    i1 = jnp.argmax(s, axis=-1).astype(jnp.int32)
    m1 = jnp.max(s, axis=-1, keepdims=True)
    s2 = jnp.where(col == i1[:, None], -1.0, s)
    i2 = jnp.argmax(s2, axis=-1).astype(jnp.int32)
    m2 = jnp.max(s2, axis=-1, keepdims=True)
    denom = m1 + m2 + 1e-20
    idx_ref[...] = jnp.concatenate([i1[None, :], i2[None, :]], 0)
    wgt_ref[...] = jnp.concatenate([(m1 / denom).T, (m2 / denom).T], 0)


def _gating(h, gw):
    return pl.pallas_call(
        _k1a_body,
        grid=(T // BM,),
        in_specs=[
            pl.BlockSpec((BM, D), lambda b: (b, 0)),
            pl.BlockSpec((E, D), lambda b: (0, 0)),
        ],
        out_specs=[
            pl.BlockSpec((K, BM), lambda b: (0, b)),
            pl.BlockSpec((K, BM), lambda b: (0, b)),
        ],
        out_shape=[
            jax.ShapeDtypeStruct((K, T), jnp.int32),
            jax.ShapeDtypeStruct((K, T), jnp.float32),
        ],
    )(h, gw)


def _k1b_body(x_ref, sw1_ref, sw3_ref, sw2_ref, ys_ref):
    hc = pl.program_id(1)
    x = x_ref[...]
    a = lax.dot_general(x, sw1_ref[...], (((1,), (1,)), ((), ())),
                        preferred_element_type=jnp.float32)
    g = lax.dot_general(x, sw3_ref[...], (((1,), (1,)), ((), ())),
                        preferred_element_type=jnp.float32)
    u = (a * jax.nn.sigmoid(a)) * g
    part = lax.dot_general(u, sw2_ref[...], (((1,), (1,)), ((), ())),
                           preferred_element_type=jnp.float32)

    @pl.when(hc == 0)
    def _init():
        ys_ref[...] = part

    @pl.when(hc != 0)
    def _acc():
        ys_ref[...] += part


def _shared_ffn(h, sw1, sw3, sw2):
    return pl.pallas_call(
        _k1b_body,
        grid=(T // BM, HC),
        in_specs=[
            pl.BlockSpec((BM, D), lambda b, hc: (b, 0)),
            pl.BlockSpec((HCS, D), lambda b, hc: (hc, 0)),
            pl.BlockSpec((HCS, D), lambda b, hc: (hc, 0)),
            pl.BlockSpec((D, HCS), lambda b, hc: (0, hc)),
        ],
        out_specs=pl.BlockSpec((BM, D), lambda b, hc: (b, 0)),
        out_shape=jax.ShapeDtypeStruct((T, D), jnp.float32),
    )(h, sw1, sw3, sw2)


def _k2_body(e_hbm, x_hbm, xs_hbm, pos_hbm, cnt_hbm,
             ev, posq, posv, cntv, rows_v, semA, semB):
    cid = lax.axis_index("c")
    sid = lax.axis_index("s")
    lane = lax.iota(jnp.int32, 16)
    sems = (semA, semB)

    pltpu.sync_copy(e_hbm, ev)
    my_first_vreg = sid * (APT // 16)

    a0 = sid * APT + cid * APW
    t0 = a0 % T
    row_cps = [
        pltpu.async_copy(x_hbm.at[pl.ds(t0 + c * 64, 64)],
                         rows_v.at[c], sems[c])
        for c in range(APW // 64)
    ]

    def _hstep(kk, carry):
        counts, prefix = carry
        v = ev[pl.ds(kk * 16, 16)]
        add = jnp.zeros((16,), jnp.int32)
        for e in range(E):
            pc = jnp.sum((v == e).astype(jnp.int32))
            add = jnp.where(lane == e, add + pc, add)
        counts = counts + add
        prefix = prefix + jnp.where(jnp.full((16,), kk < my_first_vreg),
                                    add, 0)
        return counts, prefix

    counts, prefix = lax.fori_loop(
        0, (T * K) // 16, _hstep,
        (jnp.zeros((16,), jnp.int32), jnp.zeros((16,), jnp.int32)))
    total_excl = plsc.cumsum(counts) - counts
    run = total_excl + prefix

    for kk in range(APT // 16):
        v = ev[pl.ds(sid * APT + kk * 16, 16)]
        rank = jnp.zeros((16,), jnp.int32)
        base = jnp.zeros((16,), jnp.int32)
        tot = jnp.zeros((16,), jnp.int32)
        for e in range(E):
            m = v == e
            c = plsc.cumsum(m.astype(jnp.int32))
            rank = jnp.where(m, c - 1, rank)
            run_e = jnp.sum(jnp.where(lane == e, run, 0))
            base = jnp.where(m, run_e, base)
            pc = jnp.sum(m.astype(jnp.int32))
            tot = jnp.where(lane == e, pc, tot)
        p = base + rank
        posq[kk // 4, pl.ds((kk % 4) * 16, 16)] = p
        posv[pl.ds(kk * 16, 16)] = p
        run = run + tot

    @pl.when(cid == 0)
    def _store_pos():
        pltpu.sync_copy(posv, pos_hbm.at[pl.ds(sid * APT, APT)])

    @pl.when(jnp.logical_and(cid == 0, sid == 0))
    def _store_cnt():
        cntv[...] = counts
        pltpu.sync_copy(cntv, cnt_hbm)

    sc_cps = []
    for c in range(APW // 64):
        row_cps[c].wait()
        q = cid * (APW // 64) + c
        sc_cps.append(
            pltpu.async_copy(rows_v.at[c], xs_hbm.at[posq.at[q]], sems[c]))
    for cp in sc_cps:
        cp.wait()


def _route_sc(e_flat, h):
    mesh = plsc.VectorSubcoreMesh(core_axis_name="c", subcore_axis_name="s")
    kcall = pl.kernel(
        _k2_body,
        out_type=[
            jax.ShapeDtypeStruct((T * K, D), jnp.float32),
            jax.ShapeDtypeStruct((T * K,), jnp.int32),
            jax.ShapeDtypeStruct((16,), jnp.int32),
        ],
        mesh=mesh,
        scratch_types=[
            pltpu.VMEM((T * K,), jnp.int32),
            pltpu.VMEM((APT // 64, 64), jnp.int32),
            pltpu.VMEM((APT,), jnp.int32),
            pltpu.VMEM((16,), jnp.int32),
            pltpu.VMEM((APW // 64, 64, D), jnp.float32),
            pltpu.SemaphoreType.DMA,
            pltpu.SemaphoreType.DMA,
        ],
        compiler_params=pltpu.CompilerParams(needs_layout_passes=False),
    )
    return kcall(e_flat, h)


def _make_meta(counts):
    cnt = counts[:E]
    off = jnp.cumsum(cnt) - cnt
    tiles = jnp.where(cnt > 0, (off + cnt - 1) // BM - off // BM + 1, 0)
    cum = jnp.cumsum(tiles)
    cumx = cum - tiles
    i = jnp.arange(ITEMS, dtype=jnp.int32)
    eid = jnp.minimum(jnp.searchsorted(cum, i, side="right"), E - 1)
    eid = eid.astype(jnp.int32)
    j = i - cumx[eid]
    blk = off[eid] // BM + j
    start = jnp.maximum(off[eid], BM * blk)
    end = jnp.minimum(off[eid] + cnt[eid], BM * (blk + 1))
    valid = i < cum[E - 1]
    blk = jnp.where(valid, blk, NB - 1)
    start = jnp.where(valid, start, 0)
    end = jnp.where(valid, end, 0)
    return jnp.stack([blk.astype(jnp.int32), eid,
                      start.astype(jnp.int32), end.astype(jnp.int32)])


def _k3_body(m_ref, xs_ref, w1_ref, w3_ref, w2_ref, out_ref):
    i = pl.program_id(0)
    hc = pl.program_id(1)
    blk = m_ref[0, i]
    start = m_ref[2, i]
    end = m_ref[3, i]
    prev_blk = m_ref[0, jnp.maximum(i - 1, 0)]
    first = jnp.logical_and(hc == 0,
                            jnp.logical_or(i == 0, blk != prev_blk))

    @pl.when(first)
    def _init():
        out_ref[...] = jnp.zeros_like(out_ref)

    @pl.when(start < end)
    def _compute():
        xb = xs_ref[...]
        a = lax.dot_general(xb, w1_ref[0], (((1,), (1,)), ((), ())),
                            preferred_element_type=jnp.float32)
        g = lax.dot_general(xb, w3_ref[0], (((1,), (1,)), ((), ())),
                            preferred_element_type=jnp.float32)
        u = (a * jax.nn.sigmoid(a)) * g
        r = lax.broadcasted_iota(jnp.int32, (BM, 1), 0) + BM * blk
        u = jnp.where(jnp.logical_and(r >= start, r < end), u, 0.0)
        out_ref[...] += lax.dot_general(u, w2_ref[0],
                                        (((1,), (1,)), ((), ())),
                                        preferred_element_type=jnp.float32)


def _grouped_ffn(meta, xs, w1b, w3b, w2b):
    grid_spec = pltpu.PrefetchScalarGridSpec(
        num_scalar_prefetch=1,
        grid=(ITEMS, HC),
        in_specs=[
            pl.BlockSpec((BM, D), lambda i, hc, m: (m[0, i], 0)),
            pl.BlockSpec((1, HCS, D), lambda i, hc, m: (m[1, i], hc, 0)),
            pl.BlockSpec((1, HCS, D), lambda i, hc, m: (m[1, i], hc, 0)),
            pl.BlockSpec((1, D, HCS), lambda i, hc, m: (m[1, i], 0, hc)),
        ],
        out_specs=pl.BlockSpec((BM, D), lambda i, hc, m: (m[0, i], 0)),
    )
    return pl.pallas_call(
        _k3_body,
        grid_spec=grid_spec,
        out_shape=jax.ShapeDtypeStruct((T * K, D), jnp.float32),
    )(meta, xs, w1b, w3b, w2b)


def _k4_body(os_hbm, ys_hbm, pos_hbm, wgt_hbm, y_hbm,
             p0v, p1v, w0v, w1v, r0v, r1v, ysv, yv, semA, semB, semY):
    cid = lax.axis_index("c")
    sid = lax.axis_index("s")
    wid = sid * NCORES + cid
    t0 = wid * TPW
    lane = lax.iota(jnp.int32, 16)
    sems = (semA, semB)
    NCH = TPW // 16

    pltpu.sync_copy(pos_hbm.at[0, pl.ds(t0, TPW)], p0v)
    pltpu.sync_copy(pos_hbm.at[1, pl.ds(t0, TPW)], p1v)
    pltpu.sync_copy(wgt_hbm.at[0, pl.ds(t0, TPW)], w0v)
    pltpu.sync_copy(wgt_hbm.at[1, pl.ds(t0, TPW)], w1v)

    def _fire(c):
        buf = c % 2
        idx0 = p0v[pl.ds(c * 16, 16)]
        idx1 = p1v[pl.ds(c * 16, 16)]
        return (
            pltpu.async_copy(os_hbm.at[idx0], r0v.at[buf], sems[buf]),
            pltpu.async_copy(os_hbm.at[idx1], r1v.at[buf], sems[buf]),
            pltpu.async_copy(ys_hbm.at[pl.ds(t0 + c * 16, 16)],
                             ysv.at[buf], sems[buf]),
        )

    inflight = _fire(0)
    ystores = [None, None]
    for c in range(NCH):
        buf = c % 2
        nxt = _fire(c + 1) if c + 1 < NCH else None
        for cp in inflight:
            cp.wait()
        inflight = nxt
        if ystores[buf] is not None:
            ystores[buf].wait()
        w0c = w0v[pl.ds(c * 16, 16)]
        w1c = w1v[pl.ds(c * 16, 16)]
        for j in range(16):
            w0s = jnp.sum(jnp.where(lane == j, w0c, 0.0))
            w1s = jnp.sum(jnp.where(lane == j, w1c, 0.0))

            def _dstep(dd, _, j=j, buf=buf, w0s=w0s, w1s=w1s):
                sl = pl.ds(dd * 16, 16)
                yv[buf, j, sl] = (ysv[buf, j, sl] + w0s * r0v[buf, j, sl]
                                  + w1s * r1v[buf, j, sl])
                return _

            lax.fori_loop(0, D // 16, _dstep, 0, unroll=8)
        ystores[buf] = pltpu.async_copy(
            yv.at[buf], y_hbm.at[pl.ds(t0 + c * 16, 16)], semY)
    for cp in ystores:
        if cp is not None:
            cp.wait()


def _combine_sc(out_sorted, ys, pos2, wgt2):
    mesh = plsc.VectorSubcoreMesh(core_axis_name="c", subcore_axis_name="s")
    kcall = pl.kernel(
        _k4_body,
        out_type=jax.ShapeDtypeStruct((T, D), jnp.float32),
        mesh=mesh,
        scratch_types=[
            pltpu.VMEM((TPW,), jnp.int32),
            pltpu.VMEM((TPW,), jnp.int32),
            pltpu.VMEM((TPW,), jnp.float32),
            pltpu.VMEM((TPW,), jnp.float32),
            pltpu.VMEM((2, 16, D), jnp.float32),
            pltpu.VMEM((2, 16, D), jnp.float32),
            pltpu.VMEM((2, 16, D), jnp.float32),
            pltpu.VMEM((2, 16, D), jnp.float32),
            pltpu.SemaphoreType.DMA,
            pltpu.SemaphoreType.DMA,
            pltpu.SemaphoreType.DMA,
        ],
        compiler_params=pltpu.CompilerParams(needs_layout_passes=False),
    )
    return kcall(out_sorted, ys, pos2, wgt2)


def kernel(x, gate_weight, w1, w2, w3, sw1, sw2, sw3):
    b, s, d = x.shape
    h = x.reshape(-1, d)

    eT, wgt2 = _gating(h, gate_weight)
    e_flat = eT.reshape(T * K)

    xs, pos, counts = _route_sc(e_flat, h)
    ys = _shared_ffn(h, sw1, sw3, sw2)
    meta = _make_meta(counts)
    out_sorted = _grouped_ffn(meta, xs, w1, w3, w2)
    y = _combine_sc(out_sorted, ys, pos.reshape(K, T), wgt2)
    return y.reshape(b, s, d)


# speedup vs baseline: 1.3815x; 1.0294x over previous
import functools

import jax
import jax.numpy as jnp
from jax import lax
from jax.experimental import pallas as pl
from jax.experimental.pallas import tpu as pltpu
from jax.experimental.pallas import tpu_sc as plsc

D = 768
H = 2048
E = 8
K = 2
T = 2048

BM = 256
NB = (T * K) // BM
ITEMS = NB + E - 1
HC = 2
HCS = H // HC

NCORES = 2
NSUB = 16
NW = NCORES * NSUB
APT = (T * K) // NSUB
APW = (T * K) // NW
TPW = T // NW


def _k1a_body(x_ref, gw_ref, idx_ref, wgt_ref):
    x = x_ref[...]
    logits = lax.dot_general(x, gw_ref[...],
                             (((1,), (1,)), ((), ())),
                             preferred_element_type=jnp.float32)
    col = lax.broadcasted_iota(jnp.int32, logits.shape, 1)
    m = jnp.max(logits, axis=-1, keepdims=True)
    p = jnp.exp(logits - m)
    s = p / jnp.sum(p, axis=-1, keepdims=True)
    i1 = jnp.argmax(s, axis=-1).astype(jnp.int32)
    m1 = jnp.max(s, axis=-1, keepdims=True)
    s2 = jnp.where(col == i1[:, None], -1.0, s)
    i2 = jnp.argmax(s2, axis=-1).astype(jnp.int32)
    m2 = jnp.max(s2, axis=-1, keepdims=True)
    denom = m1 + m2 + 1e-20
    idx_ref[...] = jnp.concatenate([i1[None, :], i2[None, :]], 0)
    wgt_ref[...] = jnp.concatenate([(m1 / denom).T, (m2 / denom).T], 0)


def _gating(h, gw):
    return pl.pallas_call(
        _k1a_body,
        grid=(T // BM,),
        in_specs=[
            pl.BlockSpec((BM, D), lambda b: (b, 0)),
            pl.BlockSpec((E, D), lambda b: (0, 0)),
        ],
        out_specs=[
            pl.BlockSpec((K, BM), lambda b: (0, b)),
            pl.BlockSpec((K, BM), lambda b: (0, b)),
        ],
        out_shape=[
            jax.ShapeDtypeStruct((K, T), jnp.int32),
            jax.ShapeDtypeStruct((K, T), jnp.float32),
        ],
    )(h, gw)


def _k1b_body(x_ref, sw1_ref, sw3_ref, sw2_ref, ys_ref):
    hc = pl.program_id(1)
    x = x_ref[...]
    a = lax.dot_general(x, sw1_ref[...], (((1,), (1,)), ((), ())),
                        preferred_element_type=jnp.float32)
    g = lax.dot_general(x, sw3_ref[...], (((1,), (1,)), ((), ())),
                        preferred_element_type=jnp.float32)
    u = (a * jax.nn.sigmoid(a)) * g
    part = lax.dot_general(u, sw2_ref[...], (((1,), (1,)), ((), ())),
                           preferred_element_type=jnp.float32)

    @pl.when(hc == 0)
    def _init():
        ys_ref[...] = part

    @pl.when(hc != 0)
    def _acc():
        ys_ref[...] += part


def _shared_ffn(h, sw1, sw3, sw2):
    return pl.pallas_call(
        _k1b_body,
        grid=(T // BM, HC),
        in_specs=[
            pl.BlockSpec((BM, D), lambda b, hc: (b, 0)),
            pl.BlockSpec((HCS, D), lambda b, hc: (hc, 0)),
            pl.BlockSpec((HCS, D), lambda b, hc: (hc, 0)),
            pl.BlockSpec((D, HCS), lambda b, hc: (0, hc)),
        ],
        out_specs=pl.BlockSpec((BM, D), lambda b, hc: (b, 0)),
        out_shape=jax.ShapeDtypeStruct((T, D), jnp.float32),
    )(h, sw1, sw3, sw2)


def _k2_body(e_hbm, x_hbm, xs_hbm, pos_hbm, meta_hbm,
             ev, posq, posv, metaq, rows_v, semA, semB):
    cid = lax.axis_index("c")
    sid = lax.axis_index("s")
    lane = lax.iota(jnp.int32, 16)
    sems = (semA, semB)

    pltpu.sync_copy(e_hbm, ev)
    my_first_vreg = sid * (APT // 16)

    a0 = sid * APT + cid * APW
    t0 = a0 % T
    row_cps = [
        pltpu.async_copy(x_hbm.at[pl.ds(t0 + c * 64, 64)],
                         rows_v.at[c], sems[c])
        for c in range(APW // 64)
    ]

    def _hstep(kk, carry):
        counts, prefix = carry
        v = ev[pl.ds(kk * 16, 16)]
        add = jnp.zeros((16,), jnp.int32)
        for e in range(E):
            pc = jnp.sum((v == e).astype(jnp.int32))
            add = jnp.where(lane == e, add + pc, add)
        counts = counts + add
        prefix = prefix + jnp.where(jnp.full((16,), kk < my_first_vreg),
                                    add, 0)
        return counts, prefix

    counts, prefix = lax.fori_loop(
        0, (T * K) // 16, _hstep,
        (jnp.zeros((16,), jnp.int32), jnp.zeros((16,), jnp.int32)))
    total_excl = plsc.cumsum(counts) - counts
    run = total_excl + prefix

    for kk in range(APT // 16):
        v = ev[pl.ds(sid * APT + kk * 16, 16)]
        rank = jnp.zeros((16,), jnp.int32)
        base = jnp.zeros((16,), jnp.int32)
        tot = jnp.zeros((16,), jnp.int32)
        for e in range(E):
            m = v == e
            c = plsc.cumsum(m.astype(jnp.int32))
            rank = jnp.where(m, c - 1, rank)
            run_e = jnp.sum(jnp.where(lane == e, run, 0))
            base = jnp.where(m, run_e, base)
            pc = jnp.sum(m.astype(jnp.int32))
            tot = jnp.where(lane == e, pc, tot)
        p = base + rank
        posq[kk // 4, pl.ds((kk % 4) * 16, 16)] = p
        posv[pl.ds(kk * 16, 16)] = p
        run = run + tot

    @pl.when(cid == 0)
    def _store_pos():
        pltpu.sync_copy(posv, pos_hbm.at[pl.ds(sid * APT, APT)])

    @pl.when(jnp.logical_and(cid == 0, sid == 0))
    def _meta():
        def ext(vec, e):
            return jnp.sum(jnp.where(lane == e, vec, 0))

        cnt = counts
        off = plsc.cumsum(cnt) - cnt
        tiles = jnp.where(cnt > 0,
                          ((off + cnt - 1) >> 8) - (off >> 8) + 1, 0)
        cum = plsc.cumsum(tiles)
        cumx = cum - tiles
        total = ext(cum, E - 1)
        for half in range(2):
            i = lax.iota(jnp.int32, 16) + 16 * half
            eid = jnp.zeros((16,), jnp.int32)
            for e in range(E):
                eid = eid + (ext(cum, e) <= i).astype(jnp.int32)
            eid = jnp.minimum(eid, E - 1)
            offe = jnp.zeros((16,), jnp.int32)
            cnte = jnp.zeros((16,), jnp.int32)
            cumxe = jnp.zeros((16,), jnp.int32)
            for e in range(E):
                mm = eid == e
                offe = jnp.where(mm, ext(off, e), offe)
                cnte = jnp.where(mm, ext(cnt, e), cnte)
                cumxe = jnp.where(mm, ext(cumx, e), cumxe)
            j = i - cumxe
            blk = (offe >> 8) + j
            start = jnp.maximum(offe, blk << 8)
            end = jnp.minimum(offe + cnte, (blk + 1) << 8)
            valid = i < total
            blk = jnp.where(valid, blk, NB - 1)
            start = jnp.where(valid, start, 0)
            end = jnp.where(valid, end, 0)
            sl = pl.ds(16 * half, 16)
            metaq[0, sl] = blk
            metaq[1, sl] = eid
            metaq[2, sl] = start
            metaq[3, sl] = end
        pltpu.sync_copy(metaq, meta_hbm)

    sc_cps = []
    for c in range(APW // 64):
        row_cps[c].wait()
        q = cid * (APW // 64) + c
        sc_cps.append(
            pltpu.async_copy(rows_v.at[c], xs_hbm.at[posq.at[q]], sems[c]))
    for cp in sc_cps:
        cp.wait()


def _route_sc(e_flat, h):
    mesh = plsc.VectorSubcoreMesh(core_axis_name="c", subcore_axis_name="s")
    kcall = pl.kernel(
        _k2_body,
        out_type=[
            jax.ShapeDtypeStruct((T * K, D), jnp.float32),
            jax.ShapeDtypeStruct((T * K,), jnp.int32),
            jax.ShapeDtypeStruct((4, 32), jnp.int32),
        ],
        mesh=mesh,
        scratch_types=[
            pltpu.VMEM((T * K,), jnp.int32),
            pltpu.VMEM((APT // 64, 64), jnp.int32),
            pltpu.VMEM((APT,), jnp.int32),
            pltpu.VMEM((4, 32), jnp.int32),
            pltpu.VMEM((APW // 64, 64, D), jnp.float32),
            pltpu.SemaphoreType.DMA,
            pltpu.SemaphoreType.DMA,
        ],
        compiler_params=pltpu.CompilerParams(needs_layout_passes=False),
    )
    return kcall(e_flat, h)


def _k3_body(m_ref, xs_ref, w1_ref, w3_ref, w2_ref, out_ref):
    i = pl.program_id(0)
    hc = pl.program_id(1)
    blk = m_ref[0, i]
    start = m_ref[2, i]
    end = m_ref[3, i]
    prev_blk = m_ref[0, jnp.maximum(i - 1, 0)]
    first = jnp.logical_and(hc == 0,
                            jnp.logical_or(i == 0, blk != prev_blk))

    @pl.when(first)
    def _init():
        out_ref[...] = jnp.zeros_like(out_ref)

    @pl.when(start < end)
    def _compute():
        xb = xs_ref[...]
        a = lax.dot_general(xb, w1_ref[0], (((1,), (1,)), ((), ())),
                            preferred_element_type=jnp.float32)
        g = lax.dot_general(xb, w3_ref[0], (((1,), (1,)), ((), ())),
                            preferred_element_type=jnp.float32)
        u = (a * jax.nn.sigmoid(a)) * g
        r = lax.broadcasted_iota(jnp.int32, (BM, 1), 0) + BM * blk
        u = jnp.where(jnp.logical_and(r >= start, r < end), u, 0.0)
        out_ref[...] += lax.dot_general(u, w2_ref[0],
                                        (((1,), (1,)), ((), ())),
                                        preferred_element_type=jnp.float32)


def _grouped_ffn(meta, xs, w1b, w3b, w2b):
    grid_spec = pltpu.PrefetchScalarGridSpec(
        num_scalar_prefetch=1,
        grid=(ITEMS, HC),
        in_specs=[
            pl.BlockSpec((BM, D), lambda i, hc, m: (m[0, i], 0)),
            pl.BlockSpec((1, HCS, D), lambda i, hc, m: (m[1, i], hc, 0)),
            pl.BlockSpec((1, HCS, D), lambda i, hc, m: (m[1, i], hc, 0)),
            pl.BlockSpec((1, D, HCS), lambda i, hc, m: (m[1, i], 0, hc)),
        ],
        out_specs=pl.BlockSpec((BM, D), lambda i, hc, m: (m[0, i], 0)),
    )
    return pl.pallas_call(
        _k3_body,
        grid_spec=grid_spec,
        out_shape=jax.ShapeDtypeStruct((T * K, D), jnp.float32),
    )(meta, xs, w1b, w3b, w2b)


def _k4_body(os_hbm, ys_hbm, pos_hbm, wgt_hbm, y_hbm,
             p0v, p1v, w0v, w1v, r0v, r1v, ysv, yv, semA, semB, semY):
    cid = lax.axis_index("c")
    sid = lax.axis_index("s")
    wid = sid * NCORES + cid
    t0 = wid * TPW
    lane = lax.iota(jnp.int32, 16)
    sems = (semA, semB)
    NCH = TPW // 16

    pltpu.sync_copy(pos_hbm.at[0, pl.ds(t0, TPW)], p0v)
    pltpu.sync_copy(pos_hbm.at[1, pl.ds(t0, TPW)], p1v)
    pltpu.sync_copy(wgt_hbm.at[0, pl.ds(t0, TPW)], w0v)
    pltpu.sync_copy(wgt_hbm.at[1, pl.ds(t0, TPW)], w1v)

    def _fire(c):
        buf = c % 2
        idx0 = p0v[pl.ds(c * 16, 16)]
        idx1 = p1v[pl.ds(c * 16, 16)]
        return (
            pltpu.async_copy(os_hbm.at[idx0], r0v.at[buf], sems[buf]),
            pltpu.async_copy(os_hbm.at[idx1], r1v.at[buf], sems[buf]),
            pltpu.async_copy(ys_hbm.at[pl.ds(t0 + c * 16, 16)],
                             ysv.at[buf], sems[buf]),
        )

    inflight = _fire(0)
    ystores = [None, None]
    for c in range(NCH):
        buf = c % 2
        nxt = _fire(c + 1) if c + 1 < NCH else None
        for cp in inflight:
            cp.wait()
        inflight = nxt
        if ystores[buf] is not None:
            ystores[buf].wait()
        w0c = w0v[pl.ds(c * 16, 16)]
        w1c = w1v[pl.ds(c * 16, 16)]
        for j in range(16):
            w0s = jnp.sum(jnp.where(lane == j, w0c, 0.0))
            w1s = jnp.sum(jnp.where(lane == j, w1c, 0.0))

            def _dstep(dd, _, j=j, buf=buf, w0s=w0s, w1s=w1s):
                sl = pl.ds(dd * 16, 16)
                yv[buf, j, sl] = (ysv[buf, j, sl] + w0s * r0v[buf, j, sl]
                                  + w1s * r1v[buf, j, sl])
                return _

            lax.fori_loop(0, D // 16, _dstep, 0, unroll=8)
        ystores[buf] = pltpu.async_copy(
            yv.at[buf], y_hbm.at[pl.ds(t0 + c * 16, 16)], semY)
    for cp in ystores:
        if cp is not None:
            cp.wait()


def _combine_sc(out_sorted, ys, pos2, wgt2):
    mesh = plsc.VectorSubcoreMesh(core_axis_name="c", subcore_axis_name="s")
    kcall = pl.kernel(
        _k4_body,
        out_type=jax.ShapeDtypeStruct((T, D), jnp.float32),
        mesh=mesh,
        scratch_types=[
            pltpu.VMEM((TPW,), jnp.int32),
            pltpu.VMEM((TPW,), jnp.int32),
            pltpu.VMEM((TPW,), jnp.float32),
            pltpu.VMEM((TPW,), jnp.float32),
            pltpu.VMEM((2, 16, D), jnp.float32),
            pltpu.VMEM((2, 16, D), jnp.float32),
            pltpu.VMEM((2, 16, D), jnp.float32),
            pltpu.VMEM((2, 16, D), jnp.float32),
            pltpu.SemaphoreType.DMA,
            pltpu.SemaphoreType.DMA,
            pltpu.SemaphoreType.DMA,
        ],
        compiler_params=pltpu.CompilerParams(needs_layout_passes=False),
    )
    return kcall(out_sorted, ys, pos2, wgt2)


def kernel(x, gate_weight, w1, w2, w3, sw1, sw2, sw3):
    b, s, d = x.shape
    h = x.reshape(-1, d)

    eT, wgt2 = _gating(h, gate_weight)
    e_flat = eT.reshape(T * K)

    xs, pos, meta = _route_sc(e_flat, h)
    ys = _shared_ffn(h, sw1, sw3, sw2)
    out_sorted = _grouped_ffn(meta, xs, w1, w3, w2)
    y = _combine_sc(out_sorted, ys, pos.reshape(K, T), wgt2)
    return y.reshape(b, s, d)
